# baseline (speedup 1.0000x reference)
import functools

import jax
import jax.numpy as jnp
import numpy as np
from jax.experimental import pallas as pl

N_CLASSES = 80
NA = 3
OUT = N_CLASSES + 5
STRIDE = np.array([8.0, 16.0, 32.0], dtype=np.float32)
ANCHORS = np.array([[[10, 13], [16, 30], [33, 23]],
                    [[30, 61], [62, 45], [59, 119]],
                    [[116, 90], [156, 198], [373, 326]]],
                   dtype=np.float32) / STRIDE.reshape(-1, 1, 1)

_HALF_BOX = np.where(np.arange(OUT) < 4, 0.5, 1.0).astype(np.float32)


def _head_kernel(f_ref, w_ref, b_ref, s_ref, o_ref, *, bb):
    for j in range(bb):
        fb = f_ref[j]
        for a in range(NA):
            wb = w_ref[a]
            y = jax.lax.dot_general(fb, wb, (((0,), (0,)), ((), ())),
                                    preferred_element_type=jnp.float32)
            y = y + b_ref[a]
            lane = jax.lax.broadcasted_iota(jnp.int32, y.shape, 1)
            t = jnp.tanh(y)
            s2 = 1.0 + t
            out = jnp.where(lane < 2, t + 0.5,
                            jnp.where(lane < 4, s2 * s2 * s_ref[a], y))
            o_ref[j, a] = out


@functools.partial(jax.jit, static_argnames=("bb",))
def _head_level(f, W, b, scale, bb):
    B, C, H, Wd = f.shape
    HW = H * Wd
    f = f.reshape(B, C, HW)
    Wr = W.reshape(NA, OUT, C).transpose(0, 2, 1) * _HALF_BOX[None, None, :]
    br = b.reshape(NA, 1, OUT) * _HALF_BOX[None, None, :]

    out = pl.pallas_call(
        functools.partial(_head_kernel, bb=bb),
        grid=(B // bb,),
        in_specs=[
            pl.BlockSpec((bb, C, HW), lambda bi: (bi, 0, 0)),
            pl.BlockSpec((NA, C, OUT), lambda bi: (0, 0, 0)),
            pl.BlockSpec((NA, 1, OUT), lambda bi: (0, 0, 0)),
            pl.BlockSpec((NA, 1, OUT), lambda bi: (0, 0, 0)),
        ],
        out_specs=pl.BlockSpec((bb, NA, HW, OUT), lambda bi: (bi, 0, 0, 0)),
        out_shape=jax.ShapeDtypeStruct((B, NA, HW, OUT), jnp.float32),
    )(f, Wr, br, scale)
    return out.reshape(B, NA, H, Wd, OUT)


def _scale_for_level(i):
    scale = np.ones((NA, 1, OUT), dtype=np.float32)
    scale[:, 0, 2] = ANCHORS[i][:, 0]
    scale[:, 0, 3] = ANCHORS[i][:, 1]
    return scale


_SCALES = [_scale_for_level(i) for i in range(3)]


def kernel(f0, f1, f2, W0, b0, W1, b1, W2, b2):
    outs = []
    for i, (f, W, b, bb) in enumerate([(f0, W0, b0, 1),
                                       (f1, W1, b1, 2),
                                       (f2, W2, b2, 8)]):
        outs.append(_head_level(f, W, b, _SCALES[i], bb))
    return tuple(outs)
